# Initial kernel scaffold; baseline (speedup 1.0000x reference)
import jax
import jax.numpy as jnp
from jax.experimental import pallas as pl


def kernel(w_ix, p_ix, c_ix, neg_ix, syn_ix, ms_ix, ant_ix, ma_ix, emb_i, emb_o, emb_c, W0, b0, W1, b1, Wmu, bmu, Wlv, blv):
    raise NotImplementedError("write your pallas kernel here")


# speedup vs baseline: 1.4977x; 1.4977x over previous
import functools

import jax
import jax.numpy as jnp
from jax import lax
from jax.experimental import pallas as pl
from jax.experimental.pallas import tpu as pltpu
from jax.experimental.pallas import tpu_sc as plsc

B = 4096
WIN = 20
NEG = 20
NSYN = 8
NANT = 8
D = 128
DH = 64
NC, NS, L = 2, 16, 16
NW = NC * NS
BPW = B // NW
CCB = 32
PCB = 16
SCB = 32
EPS = 1e-10

_SDS = jax.ShapeDtypeStruct


def _mesh():
    return plsc.VectorSubcoreMesh(core_axis_name="c", subcore_axis_name="s",
                                  num_cores=NC, num_subcores=NS)


_SC_PARAMS = pltpu.CompilerParams(use_tc_tiling_on_sc=False)


def _wid():
    return lax.axis_index("s") * NC + lax.axis_index("c")


def _sc_ctxt_body(cix, wix, embc, embi, ctxt_out, part_out,
                  idx_v, rows_v, ctxt_v, widx_v, wrows_v, sem):
    base = _wid() * BPW
    pltpu.sync_copy(wix.at[pl.ds(base, BPW)], widx_v)
    pltpu.async_copy(embi.at[widx_v], wrows_v, sem).wait()
    pltpu.sync_copy(wrows_v, part_out.at[pl.ds(base, BPW)])

    def chunk(ch, carry):
        pltpu.sync_copy(cix.at[pl.ds((base + ch * CCB) * WIN, CCB * WIN)], idx_v)
        pltpu.async_copy(embc.at[idx_v], rows_v, sem).wait()

        def bb(b, c2):
            r0 = b * WIN
            for l in range(D // L):
                acc = rows_v[r0, pl.ds(l * L, L)]
                for j in range(1, WIN):
                    acc = acc + rows_v[r0 + j, pl.ds(l * L, L)]
                ctxt_v[ch * CCB + b, pl.ds(l * L, L)] = acc
            return c2

        return lax.fori_loop(0, CCB, bb, carry)

    lax.fori_loop(0, BPW // CCB, chunk, 0)
    pltpu.sync_copy(ctxt_v, ctxt_out.at[pl.ds(base, BPW)])


@functools.cache
def _build_sc_ctxt():
    return pl.kernel(
        _sc_ctxt_body,
        out_type=(_SDS((B, D), jnp.float32), _SDS((B, DH), jnp.float32)),
        mesh=_mesh(),
        scratch_types=[
            pltpu.VMEM((CCB * WIN,), jnp.int32),
            pltpu.VMEM((CCB * WIN, D), jnp.float32),
            pltpu.VMEM((BPW, D), jnp.float32),
            pltpu.VMEM((BPW,), jnp.int32),
            pltpu.VMEM((BPW, DH), jnp.float32),
            pltpu.SemaphoreType.DMA,
        ],
        compiler_params=_SC_PARAMS,
    )


def _sc_ctxt(*args):
    return _build_sc_ctxt()(*args)


def _sc_dots_body(pix, nix, six, aix, embo, embi, inp, part,
                  pdot, ndot, sdot, adot,
                  inp_v, part_v, idx_v, rows_v, dot_v,
                  sidx_v, srows_v, sdot_v, sem):
    base = _wid() * BPW
    pltpu.sync_copy(inp.at[pl.ds(base, BPW)], inp_v)
    pltpu.sync_copy(part.at[pl.ds(base, BPW)], part_v)

    def pn_phase(ix, out):
        def chunk(ch, carry):
            pltpu.sync_copy(ix.at[pl.ds((base + ch * PCB) * WIN, PCB * WIN)], idx_v)
            pltpu.async_copy(embo.at[idx_v], rows_v, sem).wait()

            def bb(b, c2):
                accs = [None] * WIN
                for l in range(D // L):
                    iv = inp_v[ch * PCB + b, pl.ds(l * L, L)]
                    for j in range(WIN):
                        prod = rows_v[b * WIN + j, pl.ds(l * L, L)] * iv
                        accs[j] = prod if l == 0 else accs[j] + prod
                for j in range(WIN):
                    dot_v[b * WIN + j, :] = accs[j]
                return c2

            lax.fori_loop(0, PCB, bb, carry)
            pltpu.sync_copy(dot_v, out.at[pl.ds((base + ch * PCB) * WIN, PCB * WIN)])
            return carry

        lax.fori_loop(0, BPW // PCB, chunk, 0)

    pn_phase(pix, pdot)
    pn_phase(nix, ndot)

    def sa_phase(ix, out):
        def chunk(ch, carry):
            pltpu.sync_copy(ix.at[pl.ds((base + ch * SCB) * NSYN, SCB * NSYN)], sidx_v)
            pltpu.async_copy(embi.at[sidx_v], srows_v, sem).wait()

            def bb(b, c2):
                accs = [None] * NSYN
                for l in range(DH // L):
                    pv = part_v[ch * SCB + b, pl.ds(l * L, L)]
                    for j in range(NSYN):
                        prod = srows_v[b * NSYN + j, pl.ds(l * L, L)] * pv
                        accs[j] = prod if l == 0 else accs[j] + prod
                for j in range(NSYN):
                    sdot_v[b * NSYN + j, :] = accs[j]
                return c2

            lax.fori_loop(0, SCB, bb, carry)
            pltpu.sync_copy(sdot_v, out.at[pl.ds((base + ch * SCB) * NSYN, SCB * NSYN)])
            return carry

        lax.fori_loop(0, BPW // SCB, chunk, 0)

    sa_phase(six, sdot)
    sa_phase(aix, adot)


@functools.cache
def _build_sc_dots():
    return pl.kernel(
        _sc_dots_body,
        out_type=(_SDS((B * WIN, L), jnp.float32), _SDS((B * NEG, L), jnp.float32),
                  _SDS((B * NSYN, L), jnp.float32), _SDS((B * NANT, L), jnp.float32)),
        mesh=_mesh(),
        scratch_types=[
            pltpu.VMEM((BPW, D), jnp.float32),
            pltpu.VMEM((BPW, DH), jnp.float32),
            pltpu.VMEM((PCB * WIN,), jnp.int32),
            pltpu.VMEM((PCB * WIN, D), jnp.float32),
            pltpu.VMEM((PCB * WIN, L), jnp.float32),
            pltpu.VMEM((SCB * NSYN,), jnp.int32),
            pltpu.VMEM((SCB * NSYN, DH), jnp.float32),
            pltpu.VMEM((SCB * NSYN, L), jnp.float32),
            pltpu.SemaphoreType.DMA,
        ],
        compiler_params=_SC_PARAMS,
    )


def _sc_dots(*args):
    return _build_sc_dots()(*args)


def _tc_mlp_body(ctxt_ref, part_ref, w0, b0, w1, b1, wmu, bmu, wlv, blv, rnd,
                 inp_ref, kl_ref):
    ctxt = ctxt_ref[...]
    enc = jnp.tanh(ctxt @ w0[...] + b0[...])
    enc = jnp.tanh(enc @ w1[...] + b1[...])
    mu = enc @ wmu[...] + bmu[...]
    logvar = enc @ wlv[...] + blv[...]
    sigma = jnp.exp(logvar * 0.5)
    z = mu + sigma * rnd[...]
    inp_ref[...] = jnp.concatenate([z, part_ref[...]], axis=1)
    kl = -0.5 * jnp.sum(1.0 + logvar - mu * mu - jnp.exp(logvar))
    kl_ref[...] = kl.reshape(1, 1)


def _tc_mlp(*args):
    return pl.pallas_call(
        _tc_mlp_body,
        out_shape=(_SDS((B, D), jnp.float32), _SDS((1, 1), jnp.float32)),
    )(*args)


def _softplus(x):
    return jnp.maximum(x, 0.0) + jnp.log1p(jnp.exp(-jnp.abs(x)))


def _tc_loss_body(pd, nd, sd, ad, ms, ma, ps, ns, ss, asc):
    g = (lax.broadcasted_iota(jnp.int32, (D, D // L), 0) // L ==
         lax.broadcasted_iota(jnp.int32, (D, D // L), 1)).astype(jnp.float32)
    p = pd[...] @ g + EPS
    ps[...] = jnp.sum(_softplus(-p)).reshape(1, 1)
    n = nd[...] @ g - EPS
    ns[...] = jnp.sum(_softplus(n)).reshape(1, 1)
    s = sd[...] @ g + EPS
    ss[...] = jnp.sum(ms[...] * _softplus(-s)).reshape(1, 1)
    a = ad[...] @ g - EPS
    asc[...] = jnp.sum(ma[...] * _softplus(a)).reshape(1, 1)


def _tc_loss(*args):
    return pl.pallas_call(
        _tc_loss_body,
        out_shape=(_SDS((1, 1), jnp.float32),) * 4,
    )(*args)


def kernel(w_ix, p_ix, c_ix, neg_ix, syn_ix, ms_ix, ant_ix, ma_ix,
           emb_i, emb_o, emb_c, W0, b0, W1, b1, Wmu, bmu, Wlv, blv):
    ii = lambda a: a.reshape(-1).astype(jnp.int32)
    rnd = jax.random.normal(jax.random.key(42), (B, DH), dtype=jnp.float32)
    ctxt, part = _sc_ctxt(ii(c_ix), ii(w_ix), emb_c, emb_i)
    inp, kl_raw = _tc_mlp(ctxt, part, W0, b0.reshape(1, D), W1, b1.reshape(1, D),
                          Wmu, bmu.reshape(1, DH), Wlv, blv.reshape(1, DH), rnd)
    pdot, ndot, sdot, adot = _sc_dots(ii(p_ix), ii(neg_ix), ii(syn_ix), ii(ant_ix),
                                      emb_o, emb_i, inp, part)
    ps, ns, ss, asc = _tc_loss(pdot.reshape(B * WIN * L // D, D),
                               ndot.reshape(B * NEG * L // D, D),
                               sdot.reshape(B * NSYN * L // D, D),
                               adot.reshape(B * NANT * L // D, D),
                               ms_ix, ma_ix)
    p_score = ps[0, 0]
    n_score = ns[0, 0]
    syn_score = ss[0, 0]
    ant_score = asc[0, 0]
    kl_loss = kl_raw[0, 0] / float(WIN * NEG)
    decoder_loss = p_score + n_score + syn_score + ant_score
    loss = kl_loss + decoder_loss
    inv = 1.0 / B
    return (loss * inv, kl_loss * inv, decoder_loss * inv, p_score * inv,
            n_score * inv, syn_score * inv, ant_score * inv)


# speedup vs baseline: 1.8616x; 1.2430x over previous
import functools

import jax
import jax.numpy as jnp
from jax import lax
from jax.experimental import pallas as pl
from jax.experimental.pallas import tpu as pltpu
from jax.experimental.pallas import tpu_sc as plsc

B = 4096
WIN = 20
NEG = 20
NSYN = 8
NANT = 8
D = 128
DH = 64
NC, NS, L = 2, 16, 16
NW = NC * NS
BPW = B // NW
CCB = 16
PCB = 8
SCB = 32
EPS = 1e-10

_SDS = jax.ShapeDtypeStruct


def _mesh():
    return plsc.VectorSubcoreMesh(core_axis_name="c", subcore_axis_name="s",
                                  num_cores=NC, num_subcores=NS)


_SC_PARAMS = pltpu.CompilerParams(use_tc_tiling_on_sc=False)


def _wid():
    return lax.axis_index("s") * NC + lax.axis_index("c")


def _sc_ctxt_body(cix, wix, embc, embi, ctxt_out, part_out,
                  idx_v, rows0, rows1, ctxt_v, widx_v, wrows_v,
                  sem0, sem1, semw):
    base = _wid() * BPW
    nch = BPW // CCB
    cch = CCB * WIN
    pltpu.sync_copy(wix.at[pl.ds(base, BPW)], widx_v)
    wdesc = pltpu.async_copy(embi.at[widx_v], wrows_v, semw)
    pltpu.sync_copy(cix.at[pl.ds(base * WIN, BPW * WIN)], idx_v)

    def gather(ch, buf, sem):
        return pltpu.make_async_copy(
            embc.at[idx_v.at[pl.ds(ch * cch, cch)]], buf, sem)

    def compute(ch, buf):
        def bb(b, c2):
            r0 = b * WIN
            for l in range(D // L):
                acc = buf[r0, pl.ds(l * L, L)]
                for j in range(1, WIN):
                    acc = acc + buf[r0 + j, pl.ds(l * L, L)]
                ctxt_v[ch * CCB + b, pl.ds(l * L, L)] = acc
            return c2
        lax.fori_loop(0, CCB, bb, 0)

    gather(0, rows0, sem0).start()

    def pair(p2, _):
        ch0 = 2 * p2
        gather(ch0 + 1, rows1, sem1).start()
        gather(ch0, rows0, sem0).wait()
        compute(ch0, rows0)

        @pl.when(p2 + 1 < nch // 2)
        def _():
            gather(ch0 + 2, rows0, sem0).start()

        gather(ch0 + 1, rows1, sem1).wait()
        compute(ch0 + 1, rows1)
        return 0

    lax.fori_loop(0, nch // 2, pair, 0)
    pltpu.sync_copy(ctxt_v, ctxt_out.at[pl.ds(base, BPW)])
    wdesc.wait()
    pltpu.sync_copy(wrows_v, part_out.at[pl.ds(base, BPW)])


@functools.cache
def _build_sc_ctxt():
    return pl.kernel(
        _sc_ctxt_body,
        out_type=(_SDS((B, D), jnp.float32), _SDS((B, DH), jnp.float32)),
        mesh=_mesh(),
        scratch_types=[
            pltpu.VMEM((BPW * WIN,), jnp.int32),
            pltpu.VMEM((CCB * WIN, D), jnp.float32),
            pltpu.VMEM((CCB * WIN, D), jnp.float32),
            pltpu.VMEM((BPW, D), jnp.float32),
            pltpu.VMEM((BPW,), jnp.int32),
            pltpu.VMEM((BPW, DH), jnp.float32),
            pltpu.SemaphoreType.DMA,
            pltpu.SemaphoreType.DMA,
            pltpu.SemaphoreType.DMA,
        ],
        compiler_params=_SC_PARAMS,
    )


def _sc_ctxt(*args):
    return _build_sc_ctxt()(*args)


def _sc_dots_body(pix, nix, six, aix, embo, embi, inp, part,
                  pdot, ndot, sdot, adot,
                  inp_v, part_v, idx_v, rows0, rows1, dot_v,
                  sidx_v, srows0, srows1, sdot_v, sem0, sem1):
    base = _wid() * BPW
    pltpu.sync_copy(inp.at[pl.ds(base, BPW)], inp_v)
    pltpu.sync_copy(part.at[pl.ds(base, BPW)], part_v)

    def phase(ix, out, table, q_v, k, cb, bufs, idxbuf, dotbuf, ngroups):
        nch = BPW // cb
        chr_ = cb * k
        pltpu.sync_copy(ix.at[pl.ds(base * k, BPW * k)], idxbuf)

        def gather(ch, buf, sem):
            return pltpu.make_async_copy(
                table.at[idxbuf.at[pl.ds(ch * chr_, chr_)]], buf, sem)

        def compute(ch, buf):
            def bb(b, c2):
                accs = [None] * k
                for l in range(ngroups):
                    qv = q_v[ch * cb + b, pl.ds(l * L, L)]
                    for j in range(k):
                        prod = buf[b * k + j, pl.ds(l * L, L)] * qv
                        accs[j] = prod if l == 0 else accs[j] + prod
                for j in range(k):
                    dotbuf[b * k + j, :] = accs[j]
                return c2
            lax.fori_loop(0, cb, bb, 0)
            pltpu.sync_copy(dotbuf, out.at[pl.ds((base + ch * cb) * k, chr_)])

        gather(0, bufs[0], sem0).start()

        def pair(p2, _):
            ch0 = 2 * p2
            gather(ch0 + 1, bufs[1], sem1).start()
            gather(ch0, bufs[0], sem0).wait()
            compute(ch0, bufs[0])

            @pl.when(p2 + 1 < nch // 2)
            def _():
                gather(ch0 + 2, bufs[0], sem0).start()

            gather(ch0 + 1, bufs[1], sem1).wait()
            compute(ch0 + 1, bufs[1])
            return 0

        lax.fori_loop(0, nch // 2, pair, 0)

    phase(pix, pdot, embo, inp_v, WIN, PCB, (rows0, rows1), idx_v, dot_v, D // L)
    phase(nix, ndot, embo, inp_v, WIN, PCB, (rows0, rows1), idx_v, dot_v, D // L)
    phase(six, sdot, embi, part_v, NSYN, SCB, (srows0, srows1), sidx_v, sdot_v, DH // L)
    phase(aix, adot, embi, part_v, NSYN, SCB, (srows0, srows1), sidx_v, sdot_v, DH // L)


@functools.cache
def _build_sc_dots():
    return pl.kernel(
        _sc_dots_body,
        out_type=(_SDS((B * WIN, L), jnp.float32), _SDS((B * NEG, L), jnp.float32),
                  _SDS((B * NSYN, L), jnp.float32), _SDS((B * NANT, L), jnp.float32)),
        mesh=_mesh(),
        scratch_types=[
            pltpu.VMEM((BPW, D), jnp.float32),
            pltpu.VMEM((BPW, DH), jnp.float32),
            pltpu.VMEM((BPW * WIN,), jnp.int32),
            pltpu.VMEM((PCB * WIN, D), jnp.float32),
            pltpu.VMEM((PCB * WIN, D), jnp.float32),
            pltpu.VMEM((PCB * WIN, L), jnp.float32),
            pltpu.VMEM((BPW * NSYN,), jnp.int32),
            pltpu.VMEM((SCB * NSYN, DH), jnp.float32),
            pltpu.VMEM((SCB * NSYN, DH), jnp.float32),
            pltpu.VMEM((SCB * NSYN, L), jnp.float32),
            pltpu.SemaphoreType.DMA,
            pltpu.SemaphoreType.DMA,
        ],
        compiler_params=_SC_PARAMS,
    )


def _sc_dots(*args):
    return _build_sc_dots()(*args)


def _tc_mlp_body(ctxt_ref, part_ref, w0, b0, w1, b1, wmu, bmu, wlv, blv, rnd,
                 inp_ref, kl_ref):
    ctxt = ctxt_ref[...]
    enc = jnp.tanh(ctxt @ w0[...] + b0[...])
    enc = jnp.tanh(enc @ w1[...] + b1[...])
    mu = enc @ wmu[...] + bmu[...]
    logvar = enc @ wlv[...] + blv[...]
    sigma = jnp.exp(logvar * 0.5)
    z = mu + sigma * rnd[...]
    inp_ref[...] = jnp.concatenate([z, part_ref[...]], axis=1)
    kl = -0.5 * jnp.sum(1.0 + logvar - mu * mu - jnp.exp(logvar))
    kl_ref[...] = kl.reshape(1, 1)


def _tc_mlp(*args):
    return pl.pallas_call(
        _tc_mlp_body,
        out_shape=(_SDS((B, D), jnp.float32), _SDS((1, 1), jnp.float32)),
    )(*args)


def _softplus(x):
    return jnp.maximum(x, 0.0) + jnp.log1p(jnp.exp(-jnp.abs(x)))


def _tc_loss_body(pd, nd, sd, ad, ms, ma, ps, ns, ss, asc):
    g = (lax.broadcasted_iota(jnp.int32, (D, D // L), 0) // L ==
         lax.broadcasted_iota(jnp.int32, (D, D // L), 1)).astype(jnp.float32)
    p = pd[...] @ g + EPS
    ps[...] = jnp.sum(_softplus(-p)).reshape(1, 1)
    n = nd[...] @ g - EPS
    ns[...] = jnp.sum(_softplus(n)).reshape(1, 1)
    s = sd[...] @ g + EPS
    ss[...] = jnp.sum(ms[...] * _softplus(-s)).reshape(1, 1)
    a = ad[...] @ g - EPS
    asc[...] = jnp.sum(ma[...] * _softplus(a)).reshape(1, 1)


def _tc_loss(*args):
    return pl.pallas_call(
        _tc_loss_body,
        out_shape=(_SDS((1, 1), jnp.float32),) * 4,
    )(*args)


def kernel(w_ix, p_ix, c_ix, neg_ix, syn_ix, ms_ix, ant_ix, ma_ix,
           emb_i, emb_o, emb_c, W0, b0, W1, b1, Wmu, bmu, Wlv, blv):
    ii = lambda a: a.reshape(-1).astype(jnp.int32)
    rnd = jax.random.normal(jax.random.key(42), (B, DH), dtype=jnp.float32)
    ctxt, part = _sc_ctxt(ii(c_ix), ii(w_ix), emb_c, emb_i)
    inp, kl_raw = _tc_mlp(ctxt, part, W0, b0.reshape(1, D), W1, b1.reshape(1, D),
                          Wmu, bmu.reshape(1, DH), Wlv, blv.reshape(1, DH), rnd)
    pdot, ndot, sdot, adot = _sc_dots(ii(p_ix), ii(neg_ix), ii(syn_ix), ii(ant_ix),
                                      emb_o, emb_i, inp, part)
    ps, ns, ss, asc = _tc_loss(pdot.reshape(B * WIN * L // D, D),
                               ndot.reshape(B * NEG * L // D, D),
                               sdot.reshape(B * NSYN * L // D, D),
                               adot.reshape(B * NANT * L // D, D),
                               ms_ix, ma_ix)
    p_score = ps[0, 0]
    n_score = ns[0, 0]
    syn_score = ss[0, 0]
    ant_score = asc[0, 0]
    kl_loss = kl_raw[0, 0] / float(WIN * NEG)
    decoder_loss = p_score + n_score + syn_score + ant_score
    loss = kl_loss + decoder_loss
    inv = 1.0 / B
    return (loss * inv, kl_loss * inv, decoder_loss * inv, p_score * inv,
            n_score * inv, syn_score * inv, ant_score * inv)


# speedup vs baseline: 1.8871x; 1.0137x over previous
import functools

import jax
import jax.numpy as jnp
from jax import lax
from jax.experimental import pallas as pl
from jax.experimental.pallas import tpu as pltpu
from jax.experimental.pallas import tpu_sc as plsc

B = 4096
WIN = 20
NEG = 20
NSYN = 8
NANT = 8
D = 128
DH = 64
NC, NS, L = 2, 16, 16
NW = NC * NS
BPW = B // NW
CCB = 16
PCB = 16
SCB = 32
EPS = 1e-10

_SDS = jax.ShapeDtypeStruct


def _mesh():
    return plsc.VectorSubcoreMesh(core_axis_name="c", subcore_axis_name="s",
                                  num_cores=NC, num_subcores=NS)


_UNTILED = pltpu.CompilerParams(use_tc_tiling_on_sc=False)


def _wid():
    return lax.axis_index("s") * NC + lax.axis_index("c")


def _db_loop(nch, gather, compute, bufs, sem0, sem1):
    gather(0, bufs[0], sem0).start()

    def pair(p2, _):
        ch0 = 2 * p2
        gather(ch0 + 1, bufs[1], sem1).start()
        gather(ch0, bufs[0], sem0).wait()
        compute(ch0, bufs[0])

        @pl.when(p2 + 1 < nch // 2)
        def _():
            gather(ch0 + 2, bufs[0], sem0).start()

        gather(ch0 + 1, bufs[1], sem1).wait()
        compute(ch0 + 1, bufs[1])
        return 0

    lax.fori_loop(0, nch // 2, pair, 0)


def _sc_ctxt_body(cix, embc, ctxt_out,
                  idx_v, rows0, rows1, ctxt_v, sem0, sem1):
    base = pl.multiple_of(_wid() * BPW, BPW)
    nch = BPW // CCB
    cch = CCB * WIN
    pltpu.sync_copy(cix.at[pl.ds(pl.multiple_of(base * WIN, 8), BPW * WIN)], idx_v)

    def gather(ch, buf, sem):
        return pltpu.make_async_copy(
            embc.at[idx_v.at[pl.ds(pl.multiple_of(ch * cch, 8), cch)]], buf, sem)

    def compute(ch, buf):
        def bb(b, c2):
            r0 = b * WIN
            for l in range(D // L):
                acc = buf[r0, pl.ds(l * L, L)]
                for j in range(1, WIN):
                    acc = acc + buf[r0 + j, pl.ds(l * L, L)]
                ctxt_v[ch * CCB + b, pl.ds(l * L, L)] = acc
            return c2
        lax.fori_loop(0, CCB, bb, 0)

    _db_loop(nch, gather, compute, (rows0, rows1), sem0, sem1)
    pltpu.sync_copy(ctxt_v, ctxt_out.at[pl.ds(pl.multiple_of(base, 8), BPW)])


@functools.cache
def _build_sc_ctxt():
    return pl.kernel(
        _sc_ctxt_body,
        out_type=_SDS((B, D), jnp.float32),
        mesh=_mesh(),
        scratch_types=[
            pltpu.VMEM((BPW * WIN,), jnp.int32),
            pltpu.VMEM((CCB * WIN, D), jnp.float32),
            pltpu.VMEM((CCB * WIN, D), jnp.float32),
            pltpu.VMEM((BPW, D), jnp.float32),
            pltpu.SemaphoreType.DMA,
            pltpu.SemaphoreType.DMA,
        ],
    )


def _sc_ctxt(*args):
    return _build_sc_ctxt()(*args)


def _sc_word_body(wix, six, aix, embi, part_out, sdot, adot,
                  widx_v, wrows_v, sidx_v, srows0, srows1, sdot_v,
                  sem0, sem1, semw):
    base = _wid() * BPW
    pltpu.sync_copy(wix.at[pl.ds(base, BPW)], widx_v)
    wdesc = pltpu.async_copy(embi.at[widx_v], wrows_v, semw)

    def sa_phase(ix, out):
        nch = BPW // SCB
        chr_ = SCB * NSYN
        pltpu.sync_copy(ix.at[pl.ds(base * NSYN, BPW * NSYN)], sidx_v)

        def gather(ch, buf, sem):
            return pltpu.make_async_copy(
                embi.at[sidx_v.at[pl.ds(ch * chr_, chr_)]], buf, sem)

        def compute(ch, buf):
            def bb(b, c2):
                accs = [None] * NSYN
                for l in range(DH // L):
                    qv = wrows_v[ch * SCB + b, pl.ds(l * L, L)]
                    for j in range(NSYN):
                        prod = buf[b * NSYN + j, pl.ds(l * L, L)] * qv
                        accs[j] = prod if l == 0 else accs[j] + prod
                for j in range(NSYN):
                    sdot_v[b * NSYN + j, :] = accs[j]
                return c2
            lax.fori_loop(0, SCB, bb, 0)
            pltpu.sync_copy(sdot_v, out.at[pl.ds((base + ch * SCB) * NSYN, chr_)])

        _db_loop(nch, gather, compute, (srows0, srows1), sem0, sem1)

    wdesc.wait()
    pltpu.sync_copy(wrows_v, part_out.at[pl.ds(base, BPW)])
    sa_phase(six, sdot)
    sa_phase(aix, adot)


@functools.cache
def _build_sc_word():
    return pl.kernel(
        _sc_word_body,
        out_type=(_SDS((B, DH), jnp.float32),
                  _SDS((B * NSYN, L), jnp.float32),
                  _SDS((B * NANT, L), jnp.float32)),
        mesh=_mesh(),
        scratch_types=[
            pltpu.VMEM((BPW,), jnp.int32),
            pltpu.VMEM((BPW, DH), jnp.float32),
            pltpu.VMEM((BPW * NSYN,), jnp.int32),
            pltpu.VMEM((SCB * NSYN, DH), jnp.float32),
            pltpu.VMEM((SCB * NSYN, DH), jnp.float32),
            pltpu.VMEM((SCB * NSYN, L), jnp.float32),
            pltpu.SemaphoreType.DMA,
            pltpu.SemaphoreType.DMA,
            pltpu.SemaphoreType.DMA,
        ],
        compiler_params=_UNTILED,
    )


def _sc_word(*args):
    return _build_sc_word()(*args)


def _sc_dots_body(pix, nix, embo, inp,
                  pdot, ndot,
                  inp_v, idx_v, rows0, rows1, dot_v, sem0, sem1):
    base = pl.multiple_of(_wid() * BPW, BPW)
    pltpu.sync_copy(inp.at[pl.ds(pl.multiple_of(base, 8), BPW)], inp_v)
    nch = BPW // PCB
    chr_ = PCB * WIN
    orow = PCB * WIN * L // D

    def phase(ix, out):
        pltpu.sync_copy(ix.at[pl.ds(pl.multiple_of(base * WIN, 8), BPW * WIN)], idx_v)

        def gather(ch, buf, sem):
            return pltpu.make_async_copy(
                embo.at[idx_v.at[pl.ds(pl.multiple_of(ch * chr_, 8), chr_)]], buf, sem)

        def compute(ch, buf):
            def bb(b, c2):
                accs = [None] * WIN
                for l in range(D // L):
                    qv = inp_v[ch * PCB + b, pl.ds(l * L, L)]
                    for j in range(WIN):
                        prod = buf[b * WIN + j, pl.ds(l * L, L)] * qv
                        accs[j] = prod if l == 0 else accs[j] + prod
                for j in range(WIN):
                    f = b * WIN + j
                    dot_v[f // 8, pl.ds((f % 8) * L, L)] = accs[j]
                return c2
            lax.fori_loop(0, PCB, bb, 0)
            pltpu.sync_copy(
                dot_v,
                out.at[pl.ds(pl.multiple_of((base + ch * PCB) * WIN // 8, 8), orow)])

        _db_loop(nch, gather, compute, (rows0, rows1), sem0, sem1)

    phase(pix, pdot)
    phase(nix, ndot)


@functools.cache
def _build_sc_dots():
    return pl.kernel(
        _sc_dots_body,
        out_type=(_SDS((B * WIN * L // D, D), jnp.float32),
                  _SDS((B * NEG * L // D, D), jnp.float32)),
        mesh=_mesh(),
        scratch_types=[
            pltpu.VMEM((BPW, D), jnp.float32),
            pltpu.VMEM((BPW * WIN,), jnp.int32),
            pltpu.VMEM((PCB * WIN, D), jnp.float32),
            pltpu.VMEM((PCB * WIN, D), jnp.float32),
            pltpu.VMEM((PCB * WIN * L // D, D), jnp.float32),
            pltpu.SemaphoreType.DMA,
            pltpu.SemaphoreType.DMA,
        ],
    )


def _sc_dots(*args):
    return _build_sc_dots()(*args)


def _softplus(x):
    return jnp.maximum(x, 0.0) + jnp.log1p(jnp.exp(-jnp.abs(x)))


def _lane_group_matrix():
    return (lax.broadcasted_iota(jnp.int32, (D, D // L), 0) // L ==
            lax.broadcasted_iota(jnp.int32, (D, D // L), 1)).astype(jnp.float32)


def _tc_mlp_body(ctxt_ref, part_ref, w0, b0, w1, b1, wmu, bmu, wlv, blv, rnd,
                 sd, ad, ms, ma,
                 inp_ref, kl_ref, ss_ref, asc_ref):
    ctxt = ctxt_ref[...]
    enc = jnp.tanh(ctxt @ w0[...] + b0[...])
    enc = jnp.tanh(enc @ w1[...] + b1[...])
    mu = enc @ wmu[...] + bmu[...]
    logvar = enc @ wlv[...] + blv[...]
    sigma = jnp.exp(logvar * 0.5)
    z = mu + sigma * rnd[...]
    inp_ref[...] = jnp.concatenate([z, part_ref[...]], axis=1)
    kl = -0.5 * jnp.sum(1.0 + logvar - mu * mu - jnp.exp(logvar))
    kl_ref[...] = kl.reshape(1, 1)
    g = _lane_group_matrix()
    s = sd[...] @ g + EPS
    ss_ref[...] = jnp.sum(ms[...] * _softplus(-s)).reshape(1, 1)
    a = ad[...] @ g - EPS
    asc_ref[...] = jnp.sum(ma[...] * _softplus(a)).reshape(1, 1)


def _tc_mlp(*args):
    return pl.pallas_call(
        _tc_mlp_body,
        out_shape=(_SDS((B, D), jnp.float32), _SDS((1, 1), jnp.float32),
                   _SDS((1, 1), jnp.float32), _SDS((1, 1), jnp.float32)),
    )(*args)


def _tc_loss_body(pd, nd, ps, ns):
    g = _lane_group_matrix()
    p = pd[...] @ g + EPS
    ps[...] = jnp.sum(_softplus(-p)).reshape(1, 1)
    n = nd[...] @ g - EPS
    ns[...] = jnp.sum(_softplus(n)).reshape(1, 1)


def _tc_loss(*args):
    return pl.pallas_call(
        _tc_loss_body,
        out_shape=(_SDS((1, 1), jnp.float32),) * 2,
    )(*args)


def kernel(w_ix, p_ix, c_ix, neg_ix, syn_ix, ms_ix, ant_ix, ma_ix,
           emb_i, emb_o, emb_c, W0, b0, W1, b1, Wmu, bmu, Wlv, blv):
    ii = lambda a: a.reshape(-1).astype(jnp.int32)
    rnd = jax.random.normal(jax.random.key(42), (B, DH), dtype=jnp.float32)
    ctxt = _sc_ctxt(ii(c_ix), emb_c)
    part, sdot, adot = _sc_word(ii(w_ix), ii(syn_ix), ii(ant_ix), emb_i)
    inp, kl_raw, ss, asc = _tc_mlp(
        ctxt, part, W0, b0.reshape(1, D), W1, b1.reshape(1, D),
        Wmu, bmu.reshape(1, DH), Wlv, blv.reshape(1, DH), rnd,
        sdot.reshape(B * NSYN * L // D, D), adot.reshape(B * NANT * L // D, D),
        ms_ix, ma_ix)
    pdot, ndot = _sc_dots(ii(p_ix), ii(neg_ix), emb_o, inp)
    ps, ns = _tc_loss(pdot, ndot)
    p_score = ps[0, 0]
    n_score = ns[0, 0]
    syn_score = ss[0, 0]
    ant_score = asc[0, 0]
    kl_loss = kl_raw[0, 0] / float(WIN * NEG)
    decoder_loss = p_score + n_score + syn_score + ant_score
    loss = kl_loss + decoder_loss
    inv = 1.0 / B
    return (loss * inv, kl_loss * inv, decoder_loss * inv, p_score * inv,
            n_score * inv, syn_score * inv, ant_score * inv)


# speedup vs baseline: 2.1411x; 1.1346x over previous
import functools

import jax
import jax.numpy as jnp
from jax import lax
from jax.experimental import pallas as pl
from jax.experimental.pallas import tpu as pltpu
from jax.experimental.pallas import tpu_sc as plsc

B = 4096
WIN = 20
NEG = 20
NSYN = 8
NANT = 8
D = 128
DH = 64
NC, NS, L = 2, 16, 16
NW = NC * NS
BPW = B // NW
CCB = 16
PCB = 16
SCB = 32
EPS = 1e-10

_SDS = jax.ShapeDtypeStruct


def _mesh():
    return plsc.VectorSubcoreMesh(core_axis_name="c", subcore_axis_name="s",
                                  num_cores=NC, num_subcores=NS)


_UNTILED = pltpu.CompilerParams(use_tc_tiling_on_sc=False)


def _wid():
    return lax.axis_index("s") * NC + lax.axis_index("c")


def _db_loop(nch, gather, compute, bufs, sem0, sem1):
    gather(0, bufs[0], sem0).start()

    def pair(p2, _):
        ch0 = 2 * p2
        gather(ch0 + 1, bufs[1], sem1).start()
        gather(ch0, bufs[0], sem0).wait()
        compute(ch0, bufs[0])

        @pl.when(p2 + 1 < nch // 2)
        def _():
            gather(ch0 + 2, bufs[0], sem0).start()

        gather(ch0 + 1, bufs[1], sem1).wait()
        compute(ch0 + 1, bufs[1])
        return 0

    lax.fori_loop(0, nch // 2, pair, 0)


def _sc_ctxt_body(cix, embc, ctxt_out,
                  idx_v, rows0, rows1, ctxt_v, sem0, sem1):
    base = pl.multiple_of(_wid() * BPW, BPW)
    nch = BPW // CCB
    cch = CCB * WIN
    pltpu.sync_copy(cix.at[pl.ds(pl.multiple_of(base * WIN, 8), BPW * WIN)], idx_v)

    def gather(ch, buf, sem):
        return pltpu.make_async_copy(
            embc.at[idx_v.at[pl.ds(pl.multiple_of(ch * cch, 8), cch)]], buf, sem)

    def compute(ch, buf):
        def bb(b, c2):
            r0 = b * WIN
            for l in range(D // L):
                acc = buf[r0, pl.ds(l * L, L)]
                for j in range(1, WIN):
                    acc = acc + buf[r0 + j, pl.ds(l * L, L)]
                ctxt_v[ch * CCB + b, pl.ds(l * L, L)] = acc
            return c2
        lax.fori_loop(0, CCB, bb, 0)

    _db_loop(nch, gather, compute, (rows0, rows1), sem0, sem1)
    pltpu.sync_copy(ctxt_v, ctxt_out.at[pl.ds(pl.multiple_of(base, 8), BPW)])


@functools.cache
def _build_sc_ctxt():
    return pl.kernel(
        _sc_ctxt_body,
        out_type=_SDS((B, D), jnp.float32),
        mesh=_mesh(),
        scratch_types=[
            pltpu.VMEM((BPW * WIN,), jnp.int32),
            pltpu.VMEM((CCB * WIN, D), jnp.float32),
            pltpu.VMEM((CCB * WIN, D), jnp.float32),
            pltpu.VMEM((BPW, D), jnp.float32),
            pltpu.SemaphoreType.DMA,
            pltpu.SemaphoreType.DMA,
        ],
    )


def _sc_ctxt(*args):
    return _build_sc_ctxt()(*args)


def _sc_word_body(wix, six, aix, embi, dep, part_out, sdot, adot,
                  widx_v, wrows_v, sidx_v, srows0, srows1, sdot_v,
                  sem0, sem1, semw):
    del dep
    base = _wid() * BPW
    pltpu.sync_copy(wix.at[pl.ds(base, BPW)], widx_v)
    wdesc = pltpu.async_copy(embi.at[widx_v], wrows_v, semw)

    def sa_phase(ix, out):
        nch = BPW // SCB
        chr_ = SCB * NSYN
        pltpu.sync_copy(ix.at[pl.ds(base * NSYN, BPW * NSYN)], sidx_v)

        def gather(ch, buf, sem):
            return pltpu.make_async_copy(
                embi.at[sidx_v.at[pl.ds(ch * chr_, chr_)]], buf, sem)

        def compute(ch, buf):
            def bb(b, c2):
                accs = [None] * NSYN
                for l in range(DH // L):
                    qv = wrows_v[ch * SCB + b, pl.ds(l * L, L)]
                    for j in range(NSYN):
                        prod = buf[b * NSYN + j, pl.ds(l * L, L)] * qv
                        accs[j] = prod if l == 0 else accs[j] + prod
                for j in range(NSYN):
                    sdot_v[b * NSYN + j, :] = accs[j]
                return c2
            lax.fori_loop(0, SCB, bb, 0)
            pltpu.sync_copy(sdot_v, out.at[pl.ds((base + ch * SCB) * NSYN, chr_)])

        _db_loop(nch, gather, compute, (srows0, srows1), sem0, sem1)

    wdesc.wait()
    pltpu.sync_copy(wrows_v, part_out.at[pl.ds(base, BPW)])
    sa_phase(six, sdot)
    sa_phase(aix, adot)


@functools.cache
def _build_sc_word():
    return pl.kernel(
        _sc_word_body,
        out_type=(_SDS((B, DH), jnp.float32),
                  _SDS((B * NSYN, L), jnp.float32),
                  _SDS((B * NANT, L), jnp.float32)),
        mesh=_mesh(),
        scratch_types=[
            pltpu.VMEM((BPW,), jnp.int32),
            pltpu.VMEM((BPW, DH), jnp.float32),
            pltpu.VMEM((BPW * NSYN,), jnp.int32),
            pltpu.VMEM((SCB * NSYN, DH), jnp.float32),
            pltpu.VMEM((SCB * NSYN, DH), jnp.float32),
            pltpu.VMEM((SCB * NSYN, L), jnp.float32),
            pltpu.SemaphoreType.DMA,
            pltpu.SemaphoreType.DMA,
            pltpu.SemaphoreType.DMA,
        ],
        compiler_params=_UNTILED,
    )


def _sc_word(*args):
    return _build_sc_word()(*args)


def _sc_dots_body(pix, nix, embo, inp,
                  pdot, ndot,
                  inp_v, idx_v, rows0, rows1, dot_v, sem0, sem1):
    base = pl.multiple_of(_wid() * BPW, BPW)
    pltpu.sync_copy(inp.at[pl.ds(pl.multiple_of(base, 8), BPW)], inp_v)
    nch = BPW // PCB
    chr_ = PCB * WIN
    orow = PCB * WIN * L // D

    def phase(ix, out):
        pltpu.sync_copy(ix.at[pl.ds(pl.multiple_of(base * WIN, 8), BPW * WIN)], idx_v)

        def gather(ch, buf, sem):
            return pltpu.make_async_copy(
                embo.at[idx_v.at[pl.ds(pl.multiple_of(ch * chr_, 8), chr_)]], buf, sem)

        def compute(ch, buf):
            def bb(b, c2):
                accs = [None] * WIN
                for l in range(D // L):
                    qv = inp_v[ch * PCB + b, pl.ds(l * L, L)]
                    for j in range(WIN):
                        prod = buf[b * WIN + j, pl.ds(l * L, L)] * qv
                        accs[j] = prod if l == 0 else accs[j] + prod
                for j in range(WIN):
                    f = b * WIN + j
                    dot_v[f // 8, pl.ds((f % 8) * L, L)] = accs[j]
                return c2
            lax.fori_loop(0, PCB, bb, 0)
            pltpu.sync_copy(
                dot_v,
                out.at[pl.ds(pl.multiple_of((base + ch * PCB) * WIN // 8, 8), orow)])

        _db_loop(nch, gather, compute, (rows0, rows1), sem0, sem1)

    phase(pix, pdot)
    phase(nix, ndot)


@functools.cache
def _build_sc_dots():
    return pl.kernel(
        _sc_dots_body,
        out_type=(_SDS((B * WIN * L // D, D), jnp.float32),
                  _SDS((B * NEG * L // D, D), jnp.float32)),
        mesh=_mesh(),
        scratch_types=[
            pltpu.VMEM((BPW, D), jnp.float32),
            pltpu.VMEM((BPW * WIN,), jnp.int32),
            pltpu.VMEM((PCB * WIN, D), jnp.float32),
            pltpu.VMEM((PCB * WIN, D), jnp.float32),
            pltpu.VMEM((PCB * WIN * L // D, D), jnp.float32),
            pltpu.SemaphoreType.DMA,
            pltpu.SemaphoreType.DMA,
        ],
    )


def _sc_dots(*args):
    return _build_sc_dots()(*args)


def _softplus(x):
    return jnp.maximum(x, 0.0) + jnp.log1p(jnp.exp(-jnp.abs(x)))


def _lane_group_matrix():
    return (lax.broadcasted_iota(jnp.int32, (D, D // L), 0) // L ==
            lax.broadcasted_iota(jnp.int32, (D, D // L), 1)).astype(jnp.float32)


def _tc_mlp_body(ctxt_ref, part_ref, w0, b0, w1, b1, wmu, bmu, wlv, blv, rnd,
                 inp_ref, kl_ref):
    ctxt = ctxt_ref[...]
    enc = jnp.tanh(ctxt @ w0[...] + b0[...])
    enc = jnp.tanh(enc @ w1[...] + b1[...])
    mu = enc @ wmu[...] + bmu[...]
    logvar = enc @ wlv[...] + blv[...]
    sigma = jnp.exp(logvar * 0.5)
    z = mu + sigma * rnd[...]
    inp_ref[...] = jnp.concatenate([z, part_ref[...]], axis=1)
    kl = -0.5 * jnp.sum(1.0 + logvar - mu * mu - jnp.exp(logvar))
    kl_ref[...] = kl.reshape(1, 1)


def _tc_mlp(*args):
    return pl.pallas_call(
        _tc_mlp_body,
        out_shape=(_SDS((B, D), jnp.float32), _SDS((1, 1), jnp.float32)),
    )(*args)


def _tc_sa_body(sd, ad, ms, ma, ss_ref, asc_ref):
    g = _lane_group_matrix()
    s = sd[...] @ g + EPS
    ss_ref[...] = jnp.sum(ms[...] * _softplus(-s)).reshape(1, 1)
    a = ad[...] @ g - EPS
    asc_ref[...] = jnp.sum(ma[...] * _softplus(a)).reshape(1, 1)


def _tc_sa(*args):
    return pl.pallas_call(
        _tc_sa_body,
        out_shape=(_SDS((1, 1), jnp.float32),) * 2,
    )(*args)


def _tc_loss_body(pd, nd, ps, ns):
    g = _lane_group_matrix()
    p = pd[...] @ g + EPS
    ps[...] = jnp.sum(_softplus(-p)).reshape(1, 1)
    n = nd[...] @ g - EPS
    ns[...] = jnp.sum(_softplus(n)).reshape(1, 1)


def _tc_loss(*args):
    return pl.pallas_call(
        _tc_loss_body,
        out_shape=(_SDS((1, 1), jnp.float32),) * 2,
    )(*args)


def kernel(w_ix, p_ix, c_ix, neg_ix, syn_ix, ms_ix, ant_ix, ma_ix,
           emb_i, emb_o, emb_c, W0, b0, W1, b1, Wmu, bmu, Wlv, blv):
    ii = lambda a: a.reshape(-1).astype(jnp.int32)
    rnd = jax.random.normal(jax.random.key(42), (B, DH), dtype=jnp.float32)
    ctxt = _sc_ctxt(ii(c_ix), emb_c)
    part, sdot, adot = _sc_word(ii(w_ix), ii(syn_ix), ii(ant_ix), emb_i, ctxt)
    inp, kl_raw = _tc_mlp(
        ctxt, part, W0, b0.reshape(1, D), W1, b1.reshape(1, D),
        Wmu, bmu.reshape(1, DH), Wlv, blv.reshape(1, DH), rnd)
    pdot, ndot = _sc_dots(ii(p_ix), ii(neg_ix), emb_o, inp)
    ss, asc = _tc_sa(sdot.reshape(B * NSYN * L // D, D),
                     adot.reshape(B * NANT * L // D, D), ms_ix, ma_ix)
    ps, ns = _tc_loss(pdot, ndot)
    p_score = ps[0, 0]
    n_score = ns[0, 0]
    syn_score = ss[0, 0]
    ant_score = asc[0, 0]
    kl_loss = kl_raw[0, 0] / float(WIN * NEG)
    decoder_loss = p_score + n_score + syn_score + ant_score
    loss = kl_loss + decoder_loss
    inv = 1.0 / B
    return (loss * inv, kl_loss * inv, decoder_loss * inv, p_score * inv,
            n_score * inv, syn_score * inv, ant_score * inv)
